# Initial kernel scaffold; baseline (speedup 1.0000x reference)
import jax
import jax.numpy as jnp
from jax.experimental import pallas as pl


def kernel(x, W, b):
    raise NotImplementedError("write your pallas kernel here")


# speedup vs baseline: 1.8051x; 1.8051x over previous
import jax
import jax.numpy as jnp
from jax.experimental import pallas as pl

DIM = 2048
N_EXPERTS = 64
TOPK = 8
N_GROUPS = 8
EPG = N_EXPERTS // N_GROUPS
TOPK_GROUPS = 4
ROUTE_SCALE = 2.5
BLK = 256

_NEG = -1e30


def _gate_kernel(x_ref, w_ref, b_ref, wout_ref, iout_ref):
    x = x_ref[...]
    W = w_ref[...]
    logits = jax.lax.dot_general(
        x, W, (((1,), (1,)), ((), ())), preferred_element_type=jnp.float32
    )
    s = jax.nn.sigmoid(logits)
    sb = s + b_ref[...]

    lane = jax.lax.broadcasted_iota(jnp.int32, (1, N_EXPERTS), 1)
    gid = lane // EPG

    gs64 = jnp.zeros_like(sb)
    li = jax.lax.broadcasted_iota(jnp.int32, (1, EPG), 1)
    for g in range(N_GROUPS):
        v = sb[:, g * EPG:(g + 1) * EPG]
        m1 = jnp.max(v, axis=1, keepdims=True)
        am = jnp.argmax(v, axis=1)[:, None]
        m2 = jnp.max(jnp.where(li == am, _NEG, v), axis=1, keepdims=True)
        gs64 = jnp.where(gid == g, m1 + m2, gs64)

    keep = jnp.zeros(sb.shape, dtype=jnp.bool_)
    gw = gs64
    for _ in range(TOPK_GROUPS):
        am = jnp.argmax(gw, axis=1)[:, None]
        sel = gid == (am // EPG)
        keep = jnp.logical_or(keep, sel)
        gw = jnp.where(sel, _NEG, gw)

    masked = jnp.where(keep, sb, 0.0)

    idx_cols = []
    w_cols = []
    for _ in range(TOPK):
        am = jnp.argmax(masked, axis=1)[:, None]
        onehot = lane == am
        w_cols.append(jnp.sum(jnp.where(onehot, s, 0.0), axis=1, keepdims=True))
        idx_cols.append(am)
        masked = jnp.where(onehot, _NEG, masked)

    wts = jnp.concatenate(w_cols, axis=1)
    idxs = jnp.concatenate(idx_cols, axis=1).astype(jnp.int32)
    wts = wts / jnp.sum(wts, axis=1, keepdims=True) * ROUTE_SCALE
    wout_ref[...] = wts
    iout_ref[...] = idxs


def kernel(x, W, b):
    B = x.shape[0]
    b2 = b.reshape(1, N_EXPERTS)
    grid = (B // BLK,)
    wts, idxs = pl.pallas_call(
        _gate_kernel,
        grid=grid,
        in_specs=[
            pl.BlockSpec((BLK, DIM), lambda i: (i, 0)),
            pl.BlockSpec((N_EXPERTS, DIM), lambda i: (0, 0)),
            pl.BlockSpec((1, N_EXPERTS), lambda i: (0, 0)),
        ],
        out_specs=[
            pl.BlockSpec((BLK, TOPK), lambda i: (i, 0)),
            pl.BlockSpec((BLK, TOPK), lambda i: (i, 0)),
        ],
        out_shape=[
            jax.ShapeDtypeStruct((B, TOPK), jnp.float32),
            jax.ShapeDtypeStruct((B, TOPK), jnp.int32),
        ],
    )(x, W, b2)
    return wts.astype(x.dtype), idxs


# speedup vs baseline: 3.1372x; 1.7380x over previous
import jax
import jax.numpy as jnp
from jax.experimental import pallas as pl

DIM = 2048
N_EXPERTS = 64
TOPK = 8
N_GROUPS = 8
EPG = N_EXPERTS // N_GROUPS
TOPK_GROUPS = 4
ROUTE_SCALE = 2.5
BLK = 256

_NEG = -1e30


def _rotl(v, d):
    return jnp.concatenate([v[:, d:], v[:, :d]], axis=1)


def _gate_kernel(x_ref, w_ref, b_ref, wout_ref, iout_ref):
    x = x_ref[...]
    W = w_ref[...]
    logits = jax.lax.dot_general(
        x, W, (((1,), (1,)), ((), ())), preferred_element_type=jnp.float32
    )
    s = jax.nn.sigmoid(logits)
    sb = s + b_ref[...]

    lane = jax.lax.broadcasted_iota(jnp.int32, (1, N_EXPERTS), 1)
    gid = lane // EPG

    h2 = _rotl(sb, 4)
    hi = jnp.maximum(sb, h2)
    lo = jnp.minimum(sb, h2)
    for d in (2, 1):
        hi2 = _rotl(hi, d)
        lo2 = _rotl(lo, d)
        nlo = jnp.maximum(jnp.minimum(hi, hi2), jnp.where(hi >= hi2, lo, lo2))
        hi = jnp.maximum(hi, hi2)
        lo = nlo

    gsm = jnp.where(lane % EPG == 0, hi + lo, _NEG)

    keepf = jnp.zeros_like(sb)
    for _ in range(TOPK_GROUPS):
        am = jnp.argmax(gsm, axis=1)[:, None]
        sel = gid == (am // EPG)
        keepf = jnp.where(sel, 1.0, keepf)
        gsm = jnp.where(sel, _NEG, gsm)

    masked = sb * keepf

    idx_cols = []
    w_cols = []
    for _ in range(TOPK):
        am = jnp.argmax(masked, axis=1)[:, None]
        onehot = lane == am
        w_cols.append(jnp.sum(jnp.where(onehot, s, 0.0), axis=1, keepdims=True))
        idx_cols.append(am)
        masked = jnp.where(onehot, _NEG, masked)

    wts = jnp.concatenate(w_cols, axis=1)
    idxs = jnp.concatenate(idx_cols, axis=1).astype(jnp.int32)
    wts = wts / jnp.sum(wts, axis=1, keepdims=True) * ROUTE_SCALE
    wout_ref[...] = wts
    iout_ref[...] = idxs


def kernel(x, W, b):
    B = x.shape[0]
    b2 = b.reshape(1, N_EXPERTS)
    grid = (B // BLK,)
    wts, idxs = pl.pallas_call(
        _gate_kernel,
        grid=grid,
        in_specs=[
            pl.BlockSpec((BLK, DIM), lambda i: (i, 0)),
            pl.BlockSpec((N_EXPERTS, DIM), lambda i: (0, 0)),
            pl.BlockSpec((1, N_EXPERTS), lambda i: (0, 0)),
        ],
        out_specs=[
            pl.BlockSpec((BLK, TOPK), lambda i: (i, 0)),
            pl.BlockSpec((BLK, TOPK), lambda i: (i, 0)),
        ],
        out_shape=[
            jax.ShapeDtypeStruct((B, TOPK), jnp.float32),
            jax.ShapeDtypeStruct((B, TOPK), jnp.int32),
        ],
    )(x, W, b2)
    return wts.astype(x.dtype), idxs


# speedup vs baseline: 3.1493x; 1.0039x over previous
import jax
import jax.numpy as jnp
from jax.experimental import pallas as pl

DIM = 2048
N_EXPERTS = 64
TOPK = 8
N_GROUPS = 8
EPG = N_EXPERTS // N_GROUPS
TOPK_GROUPS = 4
ROUTE_SCALE = 2.5
BLK = 256
CHUNK = 128

_NEG = -1e30


def _rotl(v, d):
    return jnp.concatenate([v[:, d:], v[:, :d]], axis=1)


def _route(logits, b, lane, gid):
    s = jax.nn.sigmoid(logits)
    sb = s + b

    h2 = _rotl(sb, 4)
    hi = jnp.maximum(sb, h2)
    lo = jnp.minimum(sb, h2)
    for d in (2, 1):
        hi2 = _rotl(hi, d)
        lo2 = _rotl(lo, d)
        nlo = jnp.maximum(jnp.minimum(hi, hi2), jnp.where(hi >= hi2, lo, lo2))
        hi = jnp.maximum(hi, hi2)
        lo = nlo

    gsm = jnp.where(lane % EPG == 0, hi + lo, _NEG)

    keepf = jnp.zeros_like(sb)
    for t in range(TOPK_GROUPS):
        am = jnp.argmax(gsm, axis=1)[:, None]
        sel = gid == (am // EPG)
        keepf = jnp.where(sel, 1.0, keepf)
        if t < TOPK_GROUPS - 1:
            gsm = jnp.where(sel, _NEG, gsm)

    masked = sb * keepf

    idx_cols = []
    w_cols = []
    for t in range(TOPK):
        am = jnp.argmax(masked, axis=1)[:, None]
        onehot = lane == am
        w_cols.append(jnp.sum(jnp.where(onehot, s, 0.0), axis=1, keepdims=True))
        idx_cols.append(am)
        if t < TOPK - 1:
            masked = jnp.where(onehot, _NEG, masked)

    wts = jnp.concatenate(w_cols, axis=1)
    idxs = jnp.concatenate(idx_cols, axis=1).astype(jnp.int32)
    wts = wts / jnp.sum(wts, axis=1, keepdims=True) * ROUTE_SCALE
    return wts, idxs


def _gate_kernel(x_ref, w_ref, b_ref, wout_ref, iout_ref):
    x = x_ref[...]
    W = w_ref[...]
    logits = jax.lax.dot_general(
        x, W, (((1,), (1,)), ((), ())), preferred_element_type=jnp.float32
    )
    b = b_ref[...]
    lane = jax.lax.broadcasted_iota(jnp.int32, (1, N_EXPERTS), 1)
    gid = lane // EPG

    outs = [
        _route(logits[r:r + CHUNK, :], b, lane, gid)
        for r in range(0, BLK, CHUNK)
    ]
    for i, (wts, idxs) in enumerate(outs):
        wout_ref[i * CHUNK:(i + 1) * CHUNK, :] = wts
        iout_ref[i * CHUNK:(i + 1) * CHUNK, :] = idxs


def kernel(x, W, b):
    B = x.shape[0]
    b2 = b.reshape(1, N_EXPERTS)
    grid = (B // BLK,)
    wts, idxs = pl.pallas_call(
        _gate_kernel,
        grid=grid,
        in_specs=[
            pl.BlockSpec((BLK, DIM), lambda i: (i, 0)),
            pl.BlockSpec((N_EXPERTS, DIM), lambda i: (0, 0)),
            pl.BlockSpec((1, N_EXPERTS), lambda i: (0, 0)),
        ],
        out_specs=[
            pl.BlockSpec((BLK, TOPK), lambda i: (i, 0)),
            pl.BlockSpec((BLK, TOPK), lambda i: (i, 0)),
        ],
        out_shape=[
            jax.ShapeDtypeStruct((B, TOPK), jnp.float32),
            jax.ShapeDtypeStruct((B, TOPK), jnp.int32),
        ],
    )(x, W, b2)
    return wts.astype(x.dtype), idxs


# speedup vs baseline: 3.3303x; 1.0575x over previous
import jax
import jax.numpy as jnp
from jax.experimental import pallas as pl

DIM = 2048
N_EXPERTS = 64
TOPK = 8
N_GROUPS = 8
EPG = N_EXPERTS // N_GROUPS
TOPK_GROUPS = 4
ROUTE_SCALE = 2.5
BLK = 512
CHUNK = 128

_NEG = -1e30


def _rotl(v, d):
    return jnp.concatenate([v[:, d:], v[:, :d]], axis=1)


def _route(x, W, b, lane, gid):
    logits = jax.lax.dot_general(
        x, W, (((1,), (1,)), ((), ())), preferred_element_type=jnp.float32
    )
    s = jax.nn.sigmoid(logits)
    sb = s + b

    h2 = _rotl(sb, 4)
    hi = jnp.maximum(sb, h2)
    lo = jnp.minimum(sb, h2)
    for d in (2, 1):
        hi2 = _rotl(hi, d)
        lo2 = _rotl(lo, d)
        nlo = jnp.maximum(jnp.minimum(hi, hi2), jnp.where(hi >= hi2, lo, lo2))
        hi = jnp.maximum(hi, hi2)
        lo = nlo

    gsm = jnp.where(lane % EPG == 0, hi + lo, _NEG)

    keepf = jnp.zeros_like(sb)
    for t in range(TOPK_GROUPS):
        am = jnp.argmax(gsm, axis=1)[:, None]
        sel = gid == (am // EPG)
        keepf = jnp.where(sel, 1.0, keepf)
        if t < TOPK_GROUPS - 1:
            gsm = jnp.where(sel, _NEG, gsm)

    masked = sb * keepf

    idx_cols = []
    w_cols = []
    for t in range(TOPK):
        am = jnp.argmax(masked, axis=1)[:, None]
        onehot = lane == am
        w_cols.append(jnp.sum(jnp.where(onehot, s, 0.0), axis=1, keepdims=True))
        idx_cols.append(am)
        if t < TOPK - 1:
            masked = jnp.where(onehot, _NEG, masked)

    wts = jnp.concatenate(w_cols, axis=1)
    idxs = jnp.concatenate(idx_cols, axis=1).astype(jnp.int32)
    wts = wts / jnp.sum(wts, axis=1, keepdims=True) * ROUTE_SCALE
    return wts, idxs


def _gate_kernel(x_ref, w_ref, b_ref, wout_ref, iout_ref):
    W = w_ref[...]
    b = b_ref[...]
    lane = jax.lax.broadcasted_iota(jnp.int32, (1, N_EXPERTS), 1)
    gid = lane // EPG

    outs = [
        _route(x_ref[r:r + CHUNK, :], W, b, lane, gid)
        for r in range(0, BLK, CHUNK)
    ]
    for i, (wts, idxs) in enumerate(outs):
        wout_ref[i * CHUNK:(i + 1) * CHUNK, :] = wts
        iout_ref[i * CHUNK:(i + 1) * CHUNK, :] = idxs


def kernel(x, W, b):
    B = x.shape[0]
    b2 = b.reshape(1, N_EXPERTS)
    grid = (B // BLK,)
    wts, idxs = pl.pallas_call(
        _gate_kernel,
        grid=grid,
        in_specs=[
            pl.BlockSpec((BLK, DIM), lambda i: (i, 0)),
            pl.BlockSpec((N_EXPERTS, DIM), lambda i: (0, 0)),
            pl.BlockSpec((1, N_EXPERTS), lambda i: (0, 0)),
        ],
        out_specs=[
            pl.BlockSpec((BLK, TOPK), lambda i: (i, 0)),
            pl.BlockSpec((BLK, TOPK), lambda i: (i, 0)),
        ],
        out_shape=[
            jax.ShapeDtypeStruct((B, TOPK), jnp.float32),
            jax.ShapeDtypeStruct((B, TOPK), jnp.int32),
        ],
    )(x, W, b2)
    return wts.astype(x.dtype), idxs
